# baseline (speedup 1.0000x reference)
import functools

import jax
import jax.numpy as jnp
import numpy as np
from jax.experimental import pallas as pl
from jax.experimental.pallas import tpu as pltpu

_ALPHA = 0.5
_LAM = float(np.random.RandomState(0).beta(_ALPHA, 1.0 - _ALPHA))


def _mix_body(idx_ref, x_ref, xp_ref, o_ref):
    o_ref[...] = _LAM * x_ref[...] + (1.0 - _LAM) * xp_ref[...]


def kernel(x, y, index):
    B = x.shape[0]
    F = x.shape[1] * x.shape[2] * x.shape[3]
    S = F // 128
    SB = 392
    x3 = x.reshape(B, S, 128)
    grid = (B, S // SB)
    out = pl.pallas_call(
        _mix_body,
        grid_spec=pltpu.PrefetchScalarGridSpec(
            num_scalar_prefetch=1,
            grid=grid,
            in_specs=[
                pl.BlockSpec((1, SB, 128), lambda i, j, idx: (i, j, 0)),
                pl.BlockSpec((1, SB, 128), lambda i, j, idx: (idx[i], j, 0)),
            ],
            out_specs=pl.BlockSpec((1, SB, 128), lambda i, j, idx: (i, j, 0)),
        ),
        out_shape=jax.ShapeDtypeStruct((B, S, 128), jnp.float32),
    )(index, x3, x3)
    mixed = out.reshape(x.shape)
    y_b = jnp.take(y, index, axis=0)
    return (mixed, y, y_b, jnp.float32(_LAM))


# speedup vs baseline: 1.4836x; 1.4836x over previous
import functools

import jax
import jax.numpy as jnp
import numpy as np
from jax.experimental import pallas as pl
from jax.experimental.pallas import tpu as pltpu

_ALPHA = 0.5
_LAM = float(np.random.RandomState(0).beta(_ALPHA, 1.0 - _ALPHA))


def _mix_body(idx_ref, x_ref, xp_ref, o_ref):
    o_ref[...] = _LAM * x_ref[...] + (1.0 - _LAM) * xp_ref[...]


def kernel(x, y, index):
    B = x.shape[0]
    F = x.shape[1] * x.shape[2] * x.shape[3]
    S = F // 128
    SB = 1176
    x3 = x.reshape(B, S, 128)
    grid = (B, S // SB)
    out = pl.pallas_call(
        _mix_body,
        grid_spec=pltpu.PrefetchScalarGridSpec(
            num_scalar_prefetch=1,
            grid=grid,
            in_specs=[
                pl.BlockSpec((1, SB, 128), lambda i, j, idx: (i, j, 0)),
                pl.BlockSpec((1, SB, 128), lambda i, j, idx: (idx[i], j, 0)),
            ],
            out_specs=pl.BlockSpec((1, SB, 128), lambda i, j, idx: (i, j, 0)),
        ),
        out_shape=jax.ShapeDtypeStruct((B, S, 128), jnp.float32),
    )(index, x3, x3)
    mixed = out.reshape(x.shape)
    y_b = jnp.take(y, index, axis=0)
    return (mixed, y, y_b, jnp.float32(_LAM))
